# baseline (speedup 1.0000x reference)
import functools

import jax
import jax.numpy as jnp
from jax import lax
from jax.experimental import pallas as pl
from jax.experimental.pallas import tpu as pltpu
from jax.experimental.pallas import tpu_sc as plsc

N = 10000
D = 128
DAUG = 144
NT = 10008
E = 320000
NC = 2
NS = 16
CHUNK = 128
PH = 53
CH_SLOW = 53
CH_FAST = 106
EPAD = NS * (CH_SLOW + CH_FAST) * CHUNK
NACC = 10240
ROWS_SUB = NACC // NS
WB = 128

BLK_A = 1112
BLK_C = 1000


def _proj_body(x_ref, wpt_ref, bp_ref, out_ref):
    r = pl.program_id(0)
    hv = jnp.dot(x_ref[...], wpt_ref[...], preferred_element_type=jnp.float32)
    hv = jnp.maximum(hv + bp_ref[...], 0.0)
    rows = r * BLK_A + lax.broadcasted_iota(jnp.int32, (BLK_A, 1), 0)
    hb = jnp.concatenate(
        [hv, jnp.ones((BLK_A, DAUG - D), jnp.float32)], axis=1)
    out_ref[...] = jnp.where(rows < N, hb, 0.0)


_proj = pl.pallas_call(
    _proj_body,
    grid=(NT // BLK_A,),
    in_specs=[
        pl.BlockSpec((BLK_A, D), lambda r: (r, 0)),
        pl.BlockSpec((D, D), lambda r: (0, 0)),
        pl.BlockSpec((1, D), lambda r: (0, 0)),
    ],
    out_specs=pl.BlockSpec((BLK_A, DAUG), lambda r: (r, 0)),
    out_shape=jax.ShapeDtypeStruct((NT, DAUG), jnp.float32),
)


def _sc_agg(src_hbm, dst_hbm, haug_hbm, out_hbm, src_v, dst_v, rows_v, acc,
            sem):
    c = lax.axis_index("c")
    s = lax.axis_index("s")

    def zero_body(i, carry):
        for j in range(DAUG // 16):
            rows_v[i, pl.ds(j * 16, 16)] = jnp.zeros((16,), jnp.float32)
        return carry

    lax.fori_loop(0, CHUNK, zero_body, 0)
    for z in range(ROWS_SUB // WB):
        base = s * ROWS_SUB + z * WB
        pltpu.sync_copy(rows_v, acc.at[pl.ds(base, WB)])
    plsc.subcore_barrier()

    def edge_body(j, carry):
        pltpu.async_copy(haug_hbm.at[src_v.at[j]], rows_v, sem).wait()
        pltpu.sync_copy(rows_v, acc.at[dst_v.at[j]], add=True)
        return carry

    ph2 = jnp.where(c == 1, CH_SLOW - PH, PH)
    for p in range(2):
        pltpu.sync_copy(src_hbm.at[c, s, p], src_v)
        pltpu.sync_copy(dst_hbm.at[c, s, p], dst_v)
        lax.fori_loop(0, PH if p == 0 else ph2, edge_body, 0)
    plsc.subcore_barrier()

    for z in range(ROWS_SUB // WB):
        base = s * ROWS_SUB + z * WB
        pltpu.sync_copy(acc.at[pl.ds(base, WB)], rows_v)
        pltpu.sync_copy(rows_v, out_hbm.at[c, pl.ds(base, WB)])


@functools.lru_cache(maxsize=1)
def _sc_agg_call():
    mesh = plsc.VectorSubcoreMesh(core_axis_name="c", subcore_axis_name="s",
                                  num_cores=NC, num_subcores=NS)
    return pl.kernel(
        _sc_agg,
        out_type=jax.ShapeDtypeStruct((NC, NACC, DAUG), jnp.float32),
        mesh=mesh,
        scratch_types=[
            pltpu.VMEM((PH, CHUNK), jnp.int32),
            pltpu.VMEM((PH, CHUNK), jnp.int32),
            pltpu.VMEM((CHUNK, DAUG), jnp.float32),
            pltpu.VMEM_SHARED((NACC, DAUG), jnp.float32),
            pltpu.SemaphoreType.DMA,
        ],
        compiler_params=pltpu.CompilerParams(use_tc_tiling_on_sc=False),
    )


def _post_body(acc_ref, haug_ref, wlt_ref, bl_ref, wrt_ref, g_ref, be_ref,
               wot_ref, bo_ref, y_ref, gsum):
    r = pl.program_id(0)
    a0 = acc_ref[0]
    a1 = acc_ref[1]
    summed = a0[:, :D] + a1[:, :D]
    cnt = a0[:, D:D + 1] + a1[:, D:D + 1]
    agg = summed * (1.0 / jnp.maximum(cnt, 1.0))
    h = haug_ref[:, :D]
    out = (jnp.dot(agg, wlt_ref[...], preferred_element_type=jnp.float32)
           + bl_ref[...]
           + jnp.dot(h, wrt_ref[...], preferred_element_type=jnp.float32))
    mu = jnp.mean(out, axis=1, keepdims=True)
    dlt = out - mu
    var = jnp.mean(dlt * dlt, axis=1, keepdims=True)
    hn = dlt * lax.rsqrt(var + 1e-5) * g_ref[...] + be_ref[...]
    hn = jnp.maximum(hn, 0.0)
    part = jnp.sum(hn, axis=0, keepdims=True)

    @pl.when(r == 0)
    def _():
        gsum[...] = part

    @pl.when(r > 0)
    def _():
        gsum[...] = gsum[...] + part

    @pl.when(r == pl.num_programs(0) - 1)
    def _():
        g = gsum[...] * (1.0 / N)
        y_ref[...] = (jnp.dot(g, wot_ref[...],
                              preferred_element_type=jnp.float32)
                      + bo_ref[...])


_post = pl.pallas_call(
    _post_body,
    grid=(N // BLK_C,),
    in_specs=[
        pl.BlockSpec((NC, BLK_C, DAUG), lambda r: (0, r, 0)),
        pl.BlockSpec((BLK_C, DAUG), lambda r: (r, 0)),
        pl.BlockSpec((D, D), lambda r: (0, 0)),
        pl.BlockSpec((1, D), lambda r: (0, 0)),
        pl.BlockSpec((D, D), lambda r: (0, 0)),
        pl.BlockSpec((1, D), lambda r: (0, 0)),
        pl.BlockSpec((1, D), lambda r: (0, 0)),
        pl.BlockSpec((D, D), lambda r: (0, 0)),
        pl.BlockSpec((1, D), lambda r: (0, 0)),
    ],
    out_specs=pl.BlockSpec((1, D), lambda r: (0, 0)),
    out_shape=jax.ShapeDtypeStruct((1, D), jnp.float32),
    scratch_shapes=[pltpu.VMEM((1, D), jnp.float32)],
)


def kernel(x, edge_index, W_proj, b_proj, W_l, b_l, W_r, gamma, beta, W_out,
           b_out):
    haug = _proj(x, W_proj.T, b_proj.reshape(1, D))
    pad = EPAD - E

    def split(v, fill):
        vp = jnp.concatenate([v, jnp.full((pad,), fill, jnp.int32)])
        n0 = NS * CH_FAST * CHUNK
        v0 = vp[:n0].reshape(NS, 2, PH, CHUNK)
        v1 = vp[n0:].reshape(NS, CH_SLOW * CHUNK)
        v1 = jnp.pad(v1, ((0, 0), (0, (2 * PH - CH_SLOW) * CHUNK)),
                     constant_values=fill).reshape(NS, 2, PH, CHUNK)
        return jnp.stack([v0, v1])

    srcp = split(edge_index[0], N)
    dstp = split(edge_index[1], 0)
    acc = _sc_agg_call()(srcp, dstp, haug)
    return _post(acc, haug, W_l.T, b_l.reshape(1, D), W_r.T,
                 gamma.reshape(1, D), beta.reshape(1, D), W_out.T,
                 b_out.reshape(1, D))


# speedup vs baseline: 1.2718x; 1.2718x over previous
import functools

import jax
import jax.numpy as jnp
from jax import lax
from jax.experimental import pallas as pl
from jax.experimental.pallas import tpu as pltpu
from jax.experimental.pallas import tpu_sc as plsc

N = 10000
D = 128
DAUG = 144
NT = 10008
E = 320000
NC = 2
NS = 16
CHUNK = 128
PH = 51
CH_SLOW = 56
CH_FAST = 102
EPAD = NS * (CH_SLOW + CH_FAST) * CHUNK
NACC = 10240
ROWS_SUB = NACC // NS
WB = 128

BLK_A = 1112
BLK_C = 1000


def _proj_body(x_ref, wpt_ref, bp_ref, out_ref):
    r = pl.program_id(0)
    hv = jnp.dot(x_ref[...], wpt_ref[...], preferred_element_type=jnp.float32)
    hv = jnp.maximum(hv + bp_ref[...], 0.0)
    rows = r * BLK_A + lax.broadcasted_iota(jnp.int32, (BLK_A, 1), 0)
    hb = jnp.concatenate(
        [hv, jnp.ones((BLK_A, DAUG - D), jnp.float32)], axis=1)
    out_ref[...] = jnp.where(rows < N, hb, 0.0)


_proj = pl.pallas_call(
    _proj_body,
    grid=(NT // BLK_A,),
    in_specs=[
        pl.BlockSpec((BLK_A, D), lambda r: (r, 0)),
        pl.BlockSpec((D, D), lambda r: (0, 0)),
        pl.BlockSpec((1, D), lambda r: (0, 0)),
    ],
    out_specs=pl.BlockSpec((BLK_A, DAUG), lambda r: (r, 0)),
    out_shape=jax.ShapeDtypeStruct((NT, DAUG), jnp.float32),
)


def _sc_agg(src_hbm, dst_hbm, haug_hbm, out_hbm, src_v, dst_v, rows_v, acc,
            sem):
    c = lax.axis_index("c")
    s = lax.axis_index("s")

    def zero_body(i, carry):
        for j in range(DAUG // 16):
            rows_v[i, pl.ds(j * 16, 16)] = jnp.zeros((16,), jnp.float32)
        return carry

    lax.fori_loop(0, CHUNK, zero_body, 0)
    for z in range(ROWS_SUB // WB):
        base = s * ROWS_SUB + z * WB
        pltpu.sync_copy(rows_v, acc.at[pl.ds(base, WB)])
    plsc.subcore_barrier()

    def edge_body(j, carry):
        pltpu.async_copy(haug_hbm.at[src_v.at[j]], rows_v, sem).wait()
        pltpu.sync_copy(rows_v, acc.at[dst_v.at[j]], add=True)
        return carry

    ph2 = jnp.where(c == 1, CH_SLOW - PH, PH)
    for p in range(2):
        pltpu.sync_copy(src_hbm.at[c, s, p], src_v)
        pltpu.sync_copy(dst_hbm.at[c, s, p], dst_v)
        lax.fori_loop(0, PH if p == 0 else ph2, edge_body, 0)
    plsc.subcore_barrier()

    for z in range(ROWS_SUB // WB):
        base = s * ROWS_SUB + z * WB
        pltpu.sync_copy(acc.at[pl.ds(base, WB)], rows_v)
        pltpu.sync_copy(rows_v, out_hbm.at[c, pl.ds(base, WB)])


@functools.lru_cache(maxsize=1)
def _sc_agg_call():
    mesh = plsc.VectorSubcoreMesh(core_axis_name="c", subcore_axis_name="s",
                                  num_cores=NC, num_subcores=NS)
    return pl.kernel(
        _sc_agg,
        out_type=jax.ShapeDtypeStruct((NC, NACC, DAUG), jnp.float32),
        mesh=mesh,
        scratch_types=[
            pltpu.VMEM((PH, CHUNK), jnp.int32),
            pltpu.VMEM((PH, CHUNK), jnp.int32),
            pltpu.VMEM((CHUNK, DAUG), jnp.float32),
            pltpu.VMEM_SHARED((NACC, DAUG), jnp.float32),
            pltpu.SemaphoreType.DMA,
        ],
        compiler_params=pltpu.CompilerParams(use_tc_tiling_on_sc=False),
    )


def _post_body(acc_ref, haug_ref, wlt_ref, bl_ref, wrt_ref, g_ref, be_ref,
               wot_ref, bo_ref, y_ref, gsum):
    r = pl.program_id(0)
    a0 = acc_ref[0]
    a1 = acc_ref[1]
    summed = a0[:, :D] + a1[:, :D]
    cnt = a0[:, D:D + 1] + a1[:, D:D + 1]
    agg = summed * (1.0 / jnp.maximum(cnt, 1.0))
    h = haug_ref[:, :D]
    out = (jnp.dot(agg, wlt_ref[...], preferred_element_type=jnp.float32)
           + bl_ref[...]
           + jnp.dot(h, wrt_ref[...], preferred_element_type=jnp.float32))
    mu = jnp.mean(out, axis=1, keepdims=True)
    dlt = out - mu
    var = jnp.mean(dlt * dlt, axis=1, keepdims=True)
    hn = dlt * lax.rsqrt(var + 1e-5) * g_ref[...] + be_ref[...]
    hn = jnp.maximum(hn, 0.0)
    part = jnp.sum(hn, axis=0, keepdims=True)

    @pl.when(r == 0)
    def _():
        gsum[...] = part

    @pl.when(r > 0)
    def _():
        gsum[...] = gsum[...] + part

    @pl.when(r == pl.num_programs(0) - 1)
    def _():
        g = gsum[...] * (1.0 / N)
        y_ref[...] = (jnp.dot(g, wot_ref[...],
                              preferred_element_type=jnp.float32)
                      + bo_ref[...])


_post = pl.pallas_call(
    _post_body,
    grid=(N // BLK_C,),
    in_specs=[
        pl.BlockSpec((NC, BLK_C, DAUG), lambda r: (0, r, 0)),
        pl.BlockSpec((BLK_C, DAUG), lambda r: (r, 0)),
        pl.BlockSpec((D, D), lambda r: (0, 0)),
        pl.BlockSpec((1, D), lambda r: (0, 0)),
        pl.BlockSpec((D, D), lambda r: (0, 0)),
        pl.BlockSpec((1, D), lambda r: (0, 0)),
        pl.BlockSpec((1, D), lambda r: (0, 0)),
        pl.BlockSpec((D, D), lambda r: (0, 0)),
        pl.BlockSpec((1, D), lambda r: (0, 0)),
    ],
    out_specs=pl.BlockSpec((1, D), lambda r: (0, 0)),
    out_shape=jax.ShapeDtypeStruct((1, D), jnp.float32),
    scratch_shapes=[pltpu.VMEM((1, D), jnp.float32)],
)


def kernel(x, edge_index, W_proj, b_proj, W_l, b_l, W_r, gamma, beta, W_out,
           b_out):
    haug = _proj(x, W_proj.T, b_proj.reshape(1, D))
    pad = EPAD - E

    def split(v, fill):
        vp = jnp.concatenate([v, jnp.full((pad,), fill, jnp.int32)])
        n0 = NS * CH_FAST * CHUNK
        v0 = vp[:n0].reshape(NS, 2, PH, CHUNK)
        v1 = vp[n0:].reshape(NS, CH_SLOW * CHUNK)
        v1 = jnp.pad(v1, ((0, 0), (0, (2 * PH - CH_SLOW) * CHUNK)),
                     constant_values=fill).reshape(NS, 2, PH, CHUNK)
        return jnp.stack([v0, v1])

    srcp = split(edge_index[0], N)
    dstp = split(edge_index[1], 0)
    acc = _sc_agg_call()(srcp, dstp, haug)
    return _post(acc, haug, W_l.T, b_l.reshape(1, D), W_r.T,
                 gamma.reshape(1, D), beta.reshape(1, D), W_out.T,
                 b_out.reshape(1, D))
